# Initial kernel scaffold; baseline (speedup 1.0000x reference)
import jax
import jax.numpy as jnp
from jax.experimental import pallas as pl


def kernel(hidden_states, router_weight, gate_up_proj, down_proj, shared_gate_proj, shared_up_proj, shared_down_proj, shared_expert_gate):
    raise NotImplementedError("write your pallas kernel here")


# speedup vs baseline: 1.9540x; 1.9540x over previous
import functools

import jax
import jax.numpy as jnp
from jax.experimental import pallas as pl
from jax.experimental.pallas import tpu as pltpu

NUM_EXPERTS = 8
TOP_K = 2
HIDDEN = 1024
MOE_INTER = 512
SHARED_INTER = 1024
T = 2048


def _router_shared_body(x_ref, rw_ref, sg_ref, su_ref, sd_ref, seg_ref,
                        shared_out_ref, wfull_ref):
    x = x_ref[...]
    g = jax.lax.dot_general(x, sg_ref[...], (((1,), (1,)), ((), ())),
                            preferred_element_type=jnp.float32)
    u = jax.lax.dot_general(x, su_ref[...], (((1,), (1,)), ((), ())),
                            preferred_element_type=jnp.float32)
    h = jax.nn.silu(g) * u
    shared = jax.lax.dot_general(h, sd_ref[...], (((1,), (1,)), ((), ())),
                                 preferred_element_type=jnp.float32)
    tok_gate = jax.nn.sigmoid(
        jax.lax.dot_general(x, seg_ref[...], (((1,), (1,)), ((), ())),
                            preferred_element_type=jnp.float32))
    shared_out_ref[...] = tok_gate * shared

    logits = jax.lax.dot_general(x, rw_ref[...], (((1,), (1,)), ((), ())),
                                 preferred_element_type=jnp.float32)
    probs = jax.nn.softmax(logits, axis=-1)
    e_iota = jax.lax.broadcasted_iota(jnp.int32, probs.shape, 1)
    v1 = jnp.max(probs, axis=1, keepdims=True)
    i1 = jnp.argmax(probs, axis=1).reshape(-1, 1)
    masked = jnp.where(e_iota == i1, -jnp.inf, probs)
    v2 = jnp.max(masked, axis=1, keepdims=True)
    i2 = jnp.argmax(masked, axis=1).reshape(-1, 1)
    wsum = v1 + v2
    wfull_ref[...] = (jnp.where(e_iota == i1, v1 / wsum, 0.0)
                      + jnp.where(e_iota == i2, v2 / wsum, 0.0))


def _experts_body(x_ref, gup_ref, dp_ref, wfull_ref, shared_ref, out_ref):
    e = pl.program_id(0)
    x = x_ref[...]
    gu = jax.lax.dot_general(x, gup_ref[0], (((1,), (1,)), ((), ())),
                             preferred_element_type=jnp.float32)
    gate = gu[:, :MOE_INTER]
    up = gu[:, MOE_INTER:]
    h = jax.nn.silu(gate) * up
    out_e = jax.lax.dot_general(h, dp_ref[0], (((1,), (1,)), ((), ())),
                                preferred_element_type=jnp.float32)
    wf = wfull_ref[...]
    e_iota = jax.lax.broadcasted_iota(jnp.int32, wf.shape, 1)
    w_e = jnp.sum(jnp.where(e_iota == e, wf, 0.0), axis=1, keepdims=True)
    prev = jnp.where(e == 0, shared_ref[...], out_ref[...])
    out_ref[...] = prev + w_e * out_e


def kernel(hidden_states, router_weight, gate_up_proj, down_proj,
           shared_gate_proj, shared_up_proj, shared_down_proj,
           shared_expert_gate):
    B, S, H = hidden_states.shape
    x = hidden_states.reshape(-1, H)

    BT = 1024
    shared_out, wfull = pl.pallas_call(
        _router_shared_body,
        grid=(T // BT,),
        in_specs=[
            pl.BlockSpec((BT, H), lambda i: (i, 0)),
            pl.BlockSpec((NUM_EXPERTS, H), lambda i: (0, 0)),
            pl.BlockSpec((SHARED_INTER, H), lambda i: (0, 0)),
            pl.BlockSpec((SHARED_INTER, H), lambda i: (0, 0)),
            pl.BlockSpec((H, SHARED_INTER), lambda i: (0, 0)),
            pl.BlockSpec((1, H), lambda i: (0, 0)),
        ],
        out_specs=[
            pl.BlockSpec((BT, H), lambda i: (i, 0)),
            pl.BlockSpec((BT, NUM_EXPERTS), lambda i: (i, 0)),
        ],
        out_shape=[
            jax.ShapeDtypeStruct((T, H), jnp.float32),
            jax.ShapeDtypeStruct((T, NUM_EXPERTS), jnp.float32),
        ],
        compiler_params=pltpu.CompilerParams(
            dimension_semantics=("arbitrary",)),
    )(x, router_weight, shared_gate_proj, shared_up_proj, shared_down_proj,
      shared_expert_gate)

    out = pl.pallas_call(
        _experts_body,
        grid=(NUM_EXPERTS,),
        in_specs=[
            pl.BlockSpec((T, H), lambda e: (0, 0)),
            pl.BlockSpec((1, 2 * MOE_INTER, H), lambda e: (e, 0, 0)),
            pl.BlockSpec((1, H, MOE_INTER), lambda e: (e, 0, 0)),
            pl.BlockSpec((T, NUM_EXPERTS), lambda e: (0, 0)),
            pl.BlockSpec((T, H), lambda e: (0, 0)),
        ],
        out_specs=pl.BlockSpec((T, H), lambda e: (0, 0)),
        out_shape=jax.ShapeDtypeStruct((T, H), jnp.float32),
        compiler_params=pltpu.CompilerParams(
            dimension_semantics=("arbitrary",)),
    )(x, gate_up_proj, down_proj, wfull, shared_out)

    return out.reshape(B, S, H)


# speedup vs baseline: 1.9551x; 1.0006x over previous
import functools

import jax
import jax.numpy as jnp
from jax.experimental import pallas as pl
from jax.experimental.pallas import tpu as pltpu

NUM_EXPERTS = 8
TOP_K = 2
HIDDEN = 1024
MOE_INTER = 512
SHARED_INTER = 1024
T = 2048


def _router_shared_body(x_ref, rw_ref, sg_ref, su_ref, sd_ref, seg_ref,
                        shared_out_ref, wfull_ref):
    x = x_ref[...]
    xb = x.astype(jnp.bfloat16)
    g = jax.lax.dot_general(xb, sg_ref[...].astype(jnp.bfloat16),
                            (((1,), (1,)), ((), ())),
                            preferred_element_type=jnp.float32)
    u = jax.lax.dot_general(xb, su_ref[...].astype(jnp.bfloat16),
                            (((1,), (1,)), ((), ())),
                            preferred_element_type=jnp.float32)
    h = jax.nn.silu(g) * u
    shared = jax.lax.dot_general(h.astype(jnp.bfloat16),
                                 sd_ref[...].astype(jnp.bfloat16),
                                 (((1,), (1,)), ((), ())),
                                 preferred_element_type=jnp.float32)
    tok_gate = jax.nn.sigmoid(
        jax.lax.dot_general(x, seg_ref[...], (((1,), (1,)), ((), ())),
                            preferred_element_type=jnp.float32))
    shared_out_ref[...] = tok_gate * shared

    logits = jax.lax.dot_general(x, rw_ref[...], (((1,), (1,)), ((), ())),
                                 preferred_element_type=jnp.float32)
    probs = jax.nn.softmax(logits, axis=-1)
    e_iota = jax.lax.broadcasted_iota(jnp.int32, probs.shape, 1)
    v1 = jnp.max(probs, axis=1, keepdims=True)
    i1 = jnp.argmax(probs, axis=1).reshape(-1, 1)
    masked = jnp.where(e_iota == i1, -jnp.inf, probs)
    v2 = jnp.max(masked, axis=1, keepdims=True)
    i2 = jnp.argmax(masked, axis=1).reshape(-1, 1)
    wsum = v1 + v2
    wfull_ref[...] = (jnp.where(e_iota == i1, v1 / wsum, 0.0)
                      + jnp.where(e_iota == i2, v2 / wsum, 0.0))


def _experts_body(x_ref, gup_ref, dp_ref, wfull_ref, shared_ref, out_ref):
    e = pl.program_id(0)
    x = x_ref[...].astype(jnp.bfloat16)
    gu = jax.lax.dot_general(x, gup_ref[0].astype(jnp.bfloat16),
                             (((1,), (1,)), ((), ())),
                             preferred_element_type=jnp.float32)
    gate = gu[:, :MOE_INTER]
    up = gu[:, MOE_INTER:]
    h = jax.nn.silu(gate) * up
    out_e = jax.lax.dot_general(h.astype(jnp.bfloat16),
                                dp_ref[0].astype(jnp.bfloat16),
                                (((1,), (1,)), ((), ())),
                                preferred_element_type=jnp.float32)
    wf = wfull_ref[...]
    e_iota = jax.lax.broadcasted_iota(jnp.int32, wf.shape, 1)
    w_e = jnp.sum(jnp.where(e_iota == e, wf, 0.0), axis=1, keepdims=True)
    prev = jnp.where(e == 0, shared_ref[...], out_ref[...])
    out_ref[...] = prev + w_e * out_e


def kernel(hidden_states, router_weight, gate_up_proj, down_proj,
           shared_gate_proj, shared_up_proj, shared_down_proj,
           shared_expert_gate):
    B, S, H = hidden_states.shape
    x = hidden_states.reshape(-1, H)

    BT = 1024
    shared_out, wfull = pl.pallas_call(
        _router_shared_body,
        grid=(T // BT,),
        in_specs=[
            pl.BlockSpec((BT, H), lambda i: (i, 0)),
            pl.BlockSpec((NUM_EXPERTS, H), lambda i: (0, 0)),
            pl.BlockSpec((SHARED_INTER, H), lambda i: (0, 0)),
            pl.BlockSpec((SHARED_INTER, H), lambda i: (0, 0)),
            pl.BlockSpec((H, SHARED_INTER), lambda i: (0, 0)),
            pl.BlockSpec((1, H), lambda i: (0, 0)),
        ],
        out_specs=[
            pl.BlockSpec((BT, H), lambda i: (i, 0)),
            pl.BlockSpec((BT, NUM_EXPERTS), lambda i: (i, 0)),
        ],
        out_shape=[
            jax.ShapeDtypeStruct((T, H), jnp.float32),
            jax.ShapeDtypeStruct((T, NUM_EXPERTS), jnp.float32),
        ],
        compiler_params=pltpu.CompilerParams(
            dimension_semantics=("arbitrary",)),
    )(x, router_weight, shared_gate_proj, shared_up_proj, shared_down_proj,
      shared_expert_gate)

    out = pl.pallas_call(
        _experts_body,
        grid=(NUM_EXPERTS,),
        in_specs=[
            pl.BlockSpec((T, H), lambda e: (0, 0)),
            pl.BlockSpec((1, 2 * MOE_INTER, H), lambda e: (e, 0, 0)),
            pl.BlockSpec((1, H, MOE_INTER), lambda e: (e, 0, 0)),
            pl.BlockSpec((T, NUM_EXPERTS), lambda e: (0, 0)),
            pl.BlockSpec((T, H), lambda e: (0, 0)),
        ],
        out_specs=pl.BlockSpec((T, H), lambda e: (0, 0)),
        out_shape=jax.ShapeDtypeStruct((T, H), jnp.float32),
        compiler_params=pltpu.CompilerParams(
            dimension_semantics=("arbitrary",)),
    )(x, gate_up_proj, down_proj, wfull, shared_out)

    return out.reshape(B, S, H)
